# baseline (speedup 1.0000x reference)
import functools

import jax
import jax.numpy as jnp
from jax import lax
from jax.experimental import pallas as pl
from jax.experimental.pallas import tpu as pltpu
from jax.experimental.pallas import tpu_sc as plsc

_LANES = 16
_TILES = 16
_BN = 400


def _zero_rows(ref, nrows, ncols):
    zv = jnp.zeros((_LANES,), jnp.float32)

    def body(i, _):
        for t in range(ncols // _LANES):
            ref[i, pl.ds(t * _LANES, _LANES)] = zv
        return 0

    lax.fori_loop(0, nrows, body, 0)


def _node_partition(n):
    npt = (n // _TILES) & ~7
    tail = n - npt * _TILES
    return npt, tail


def _sliced_copy(src_fn, dst_fn, total, chunk):
    for b in range(total // chunk):
        pltpu.sync_copy(src_fn(b * chunk, chunk), dst_fn(b * chunk, chunk))
    rem = total % chunk
    if rem:
        off = (total // chunk) * chunk
        pltpu.sync_copy(src_fn(off, rem), dst_fn(off, rem))


def _fill_rows(ref, nrows, ncols, value):
    fv = jnp.full((_LANES,), value, jnp.float32)

    def body(i, _):
        for t in range(ncols // _LANES):
            ref[i, pl.ds(t * _LANES, _LANES)] = fv
        return 0

    lax.fori_loop(0, nrows, body, 0)


@functools.partial(jax.jit, static_argnames=("n", "e"))
def _sc_deg(dst, *, n, e):
    k = 80
    assert e % (2 * _TILES * k) == 0
    ept = e // (2 * _TILES)
    nblk = ept // k
    npt, tail = _node_partition(n)
    mesh = plsc.VectorSubcoreMesh(core_axis_name="c", subcore_axis_name="s",
                                  num_cores=2, num_subcores=_TILES)

    @functools.partial(
        pl.kernel,
        out_type=jax.ShapeDtypeStruct((2 * n, 128), jnp.float32),
        mesh=mesh,
        scratch_types=[
            pltpu.VMEM((k, 128), jnp.float32),
            pltpu.VMEM((k, 128), jnp.float32),
            pltpu.VMEM((8 * k,), jnp.int32),
            pltpu.VMEM_SHARED((n, 128), jnp.float32),
            pltpu.SemaphoreType.DMA((8,)),
            pltpu.SemaphoreType.DMA((4,)),
        ],
    )
    def deg_kernel(dst_hbm, out_hbm, ones, zrows, didx, acc, isem, ssem):
        c = lax.axis_index("c")
        s = lax.axis_index("s")
        _fill_rows(ones, k, 128, 1.0)
        _zero_rows(zrows, k, 128)
        tbase = c * (e // 2) + s * ept

        base = s * npt
        _sliced_copy(lambda o, sz: zrows.at[pl.ds(0, sz)],
                     lambda o, sz: acc.at[pl.ds(base + o, sz)], npt, k)

        @pl.when(s == _TILES - 1)
        def _():
            if tail:
                pltpu.sync_copy(zrows.at[pl.ds(0, tail)],
                                acc.at[pl.ds(npt * _TILES, tail)])

        plsc.subcore_barrier()

        def didx_at(slot):
            return didx.at[pl.ds(slot * k, k)]

        def fire_idx(b):
            pltpu.async_copy(dst_hbm.at[pl.ds(tbase + b * k, k)],
                             didx_at(lax.rem(b, 8)), isem.at[lax.rem(b, 8)])

        def wait_idx(b):
            pltpu.make_async_copy(dst_hbm.at[pl.ds(tbase, k)],
                                  didx_at(lax.rem(b, 8)),
                                  isem.at[lax.rem(b, 8)]).wait()

        def wait_scat(b):
            pltpu.make_async_copy(ones, acc.at[didx_at(lax.rem(b, 8))],
                                  ssem.at[lax.rem(b, 4)]).wait()

        fire_idx(jnp.int32(0))
        fire_idx(jnp.int32(1))
        fire_idx(jnp.int32(2))

        def eblk(b, _):
            wait_idx(b)

            @pl.when(b >= 4)
            def _():
                wait_scat(b - 4)

            pltpu.async_copy(ones, acc.at[didx_at(lax.rem(b, 8))],
                             ssem.at[lax.rem(b, 4)], add=True)

            @pl.when(b + 3 < nblk)
            def _():
                fire_idx(b + 3)

            return 0

        lax.fori_loop(0, nblk, eblk, 0)
        for t in range(4):
            wait_scat(jnp.int32(nblk - 4 + t))
        plsc.subcore_barrier()
        pltpu.sync_copy(acc.at[pl.ds(base, npt)],
                        out_hbm.at[pl.ds(c * n + base, npt)])

        @pl.when(s == _TILES - 1)
        def _():
            if tail:
                pltpu.sync_copy(acc.at[pl.ds(npt * _TILES, tail)],
                                out_hbm.at[pl.ds(c * n + npt * _TILES, tail)])

    return deg_kernel(dst)


_K = 64
_NSLOT = 4
_SLAB = 8
_IRING = 3


@functools.partial(jax.jit, static_argnames=("n", "e", "nc"))
def _sc_prop(hs_flat, gsrc, dst, *, n, e, nc):
    k = _K
    ept = e // _TILES
    nblk = ept // k
    assert e % (_TILES * k) == 0 and nblk % _SLAB == 0
    nslab = nblk // _SLAB
    assert nslab >= 2
    slab_i = _SLAB * k
    npt, tail = _node_partition(n)
    cpq = nc // 2
    mesh = plsc.VectorSubcoreMesh(core_axis_name="c", subcore_axis_name="s",
                                  num_cores=2, num_subcores=_TILES)

    @functools.partial(
        pl.kernel,
        out_type=jax.ShapeDtypeStruct((nc * n, 128), jnp.float32),
        mesh=mesh,
        scratch_types=[
            [pltpu.VMEM((k, 128), jnp.float32) for _ in range(_NSLOT)],
            pltpu.VMEM((_IRING * slab_i,), jnp.int32),
            pltpu.VMEM((_IRING * slab_i,), jnp.int32),
            pltpu.VMEM_SHARED((n + 8, 128), jnp.float32),
            pltpu.SemaphoreType.DMA((_NSLOT,)),
            pltpu.SemaphoreType.DMA((_NSLOT,)),
            pltpu.SemaphoreType.DMA((_IRING,)),
            pltpu.SemaphoreType.DMA((_IRING,)),
        ],
    )
    def prop_kernel(hs_hbm, gsrc_hbm, dst_hbm, out_hbm,
                    rows, gbuf, dbuf, acc, gsem, ssem, gisem, disem):
        c = lax.axis_index("c")
        s = lax.axis_index("s")
        base = s * npt
        tbase = s * ept

        def gidx_at(ring, r):
            return gbuf.at[pl.ds(ring * slab_i + r * k, k)]

        def didx_at(ring, r):
            return dbuf.at[pl.ds(ring * slab_i + r * k, k)]

        def fire_gather(slot, ring, r):
            pltpu.async_copy(hs_hbm.at[gidx_at(ring, r)], rows[slot],
                             gsem.at[slot])

        def wait_gather(slot, ring, r):
            pltpu.make_async_copy(hs_hbm.at[gidx_at(ring, r)], rows[slot],
                                  gsem.at[slot]).wait()

        def fire_scatter(slot, ring, r):
            pltpu.async_copy(rows[slot], acc.at[didx_at(ring, r)],
                             ssem.at[slot], add=True)

        def wait_scatter(slot, ring, r):
            pltpu.make_async_copy(rows[slot], acc.at[didx_at(ring, r)],
                                  ssem.at[slot]).wait()

        for j in range(cpq):
            q = c * cpq + j
            qrow = q * n
            qsrc = q * e

            _zero_rows(rows[0], k, 128)
            _sliced_copy(lambda o, sz: rows[0].at[pl.ds(0, sz)],
                         lambda o, sz: acc.at[pl.ds(base + o, sz)], npt, k)

            @pl.when(s == _TILES - 1)
            def _():
                if tail:
                    pltpu.sync_copy(rows[0].at[pl.ds(0, tail)],
                                    acc.at[pl.ds(npt * _TILES, tail)])

            plsc.subcore_barrier()

            pltpu.sync_copy(gsrc_hbm.at[pl.ds(qsrc + tbase, slab_i)],
                            gbuf.at[pl.ds(0, slab_i)])
            pltpu.sync_copy(dst_hbm.at[pl.ds(tbase, slab_i)],
                            dbuf.at[pl.ds(0, slab_i)])
            fire_gather(0, 0, 0)
            fire_gather(1, 0, 1)

            def slab(g, _):
                ring = lax.rem(g, _IRING)
                nring = lax.rem(g + 1, _IRING)

                @pl.when(g < nslab - 1)
                def _():
                    off = (g + 1) * slab_i
                    pltpu.async_copy(
                        gsrc_hbm.at[pl.ds(qsrc + tbase + off, slab_i)],
                        gbuf.at[pl.ds(nring * slab_i, slab_i)],
                        gisem.at[nring])
                    pltpu.async_copy(
                        dst_hbm.at[pl.ds(tbase + off, slab_i)],
                        dbuf.at[pl.ds(nring * slab_i, slab_i)],
                        disem.at[nring])

                for jj in range(_SLAB):
                    t = jj % _NSLOT
                    sp = (t + 2) % _NSLOT
                    wait_gather(t, ring, jj)
                    fire_scatter(t, ring, jj)
                    if jj < 2:
                        @pl.when(g > 0)
                        def _():
                            wait_scatter(sp, ring, jj)
                        fire_gather(sp, ring, jj + 2)
                    elif jj < _SLAB - 2:
                        wait_scatter(sp, ring, jj)
                        fire_gather(sp, ring, jj + 2)
                    else:
                        wait_scatter(sp, ring, jj)
                        if jj == _SLAB - 2:
                            @pl.when(g < nslab - 1)
                            def _():
                                pltpu.make_async_copy(
                                    gsrc_hbm.at[pl.ds(qsrc + tbase, slab_i)],
                                    gbuf.at[pl.ds(nring * slab_i, slab_i)],
                                    gisem.at[nring]).wait()
                                pltpu.make_async_copy(
                                    dst_hbm.at[pl.ds(tbase, slab_i)],
                                    dbuf.at[pl.ds(nring * slab_i, slab_i)],
                                    disem.at[nring]).wait()

                        @pl.when(g < nslab - 1)
                        def _():
                            fire_gather(sp, nring, jj - (_SLAB - 2))
                return 0

            lax.fori_loop(0, nslab, slab, 0)
            wait_scatter(2, 0, 0)
            wait_scatter(3, 0, 0)
            plsc.subcore_barrier()
            pltpu.sync_copy(acc.at[pl.ds(base, npt)],
                            out_hbm.at[pl.ds(qrow + base, npt)])

            @pl.when(s == _TILES - 1)
            def _():
                if tail:
                    pltpu.sync_copy(acc.at[pl.ds(npt * _TILES, tail)],
                                    out_hbm.at[pl.ds(qrow + npt * _TILES, tail)])

            plsc.subcore_barrier()

    return prop_kernel(hs_flat, gsrc, dst)


def _tc_input(x, w0c, deg16, *, n, din):
    def body(x_ref, w_ref, deg_ref, hs_ref, dinv_ref):
        d = deg_ref[0, :, 0:1] + deg_ref[1, :, 0:1]
        dv = lax.rsqrt(jnp.maximum(d + 1.0, 1.0))
        h = jnp.dot(x_ref[...], w_ref[...], preferred_element_type=jnp.float32)
        hs_ref[0] = h * dv
        dinv_ref[...] = dv

    grid = (n // _BN, 4)
    return pl.pallas_call(
        body,
        grid=grid,
        in_specs=[
            pl.BlockSpec((_BN, din), lambda i, q: (i, 0)),
            pl.BlockSpec((din, 128), lambda i, q: (0, q)),
            pl.BlockSpec((2, _BN, 128), lambda i, q: (0, i, 0)),
        ],
        out_specs=[
            pl.BlockSpec((1, _BN, 128), lambda i, q: (q, i, 0)),
            pl.BlockSpec((_BN, 1), lambda i, q: (i, 0)),
        ],
        out_shape=[
            jax.ShapeDtypeStruct((4, n, 128), jnp.float32),
            jax.ShapeDtypeStruct((n, 1), jnp.float32),
        ],
    )(x, w0c, deg16)


def _tc_hidden(s4, hs4, dinv, bias4, bk, *, n):
    def body(s_ref, hs_ref, dinv_ref, b_ref, w_ref, out_ref):
        dv = dinv_ref[...]
        acc = jnp.zeros((_BN, 128), jnp.float32)
        for qi in range(4):
            f = dv * (s_ref[qi] + hs_ref[qi]) + b_ref[qi]
            acc += jnp.dot(f, w_ref[pl.ds(qi * 128, 128), :],
                           preferred_element_type=jnp.float32)
        out_ref[0] = acc * dv

    grid = (n // _BN, 4)
    return pl.pallas_call(
        body,
        grid=grid,
        in_specs=[
            pl.BlockSpec((4, _BN, 128), lambda i, qo: (0, i, 0)),
            pl.BlockSpec((4, _BN, 128), lambda i, qo: (0, i, 0)),
            pl.BlockSpec((_BN, 1), lambda i, qo: (i, 0)),
            pl.BlockSpec((4, 1, 128), lambda i, qo: (0, 0, 0)),
            pl.BlockSpec((512, 128), lambda i, qo: (0, qo)),
        ],
        out_specs=pl.BlockSpec((1, _BN, 128), lambda i, qo: (qo, i, 0)),
        out_shape=jax.ShapeDtypeStruct((4, n, 128), jnp.float32),
    )(s4, hs4, dinv, bias4, bk)


def _tc_output(s2, hs2, dinv, bias2, wo, bo, *, n, dout):
    def body(s_ref, hs_ref, dinv_ref, b_ref, w_ref, bo_ref, out_ref):
        dv = dinv_ref[...]
        acc = jnp.zeros((_BN, dout), jnp.float32) + bo_ref[...]
        for qi in range(2):
            f = dv * (s_ref[qi] + hs_ref[qi]) + b_ref[qi]
            acc += jnp.dot(f, w_ref[pl.ds(qi * 128, 128), :],
                           preferred_element_type=jnp.float32)
        out_ref[...] = acc

    grid = (n // _BN,)
    return pl.pallas_call(
        body,
        grid=grid,
        in_specs=[
            pl.BlockSpec((2, _BN, 128), lambda i: (0, i, 0)),
            pl.BlockSpec((2, _BN, 128), lambda i: (0, i, 0)),
            pl.BlockSpec((_BN, 1), lambda i: (i, 0)),
            pl.BlockSpec((2, 1, 128), lambda i: (0, 0, 0)),
            pl.BlockSpec((256, dout), lambda i: (0, 0)),
            pl.BlockSpec((1, dout), lambda i: (0, 0)),
        ],
        out_specs=pl.BlockSpec((_BN, dout), lambda i: (i, 0)),
        out_shape=jax.ShapeDtypeStruct((n, dout), jnp.float32),
    )(s2, hs2, dinv, bias2, wo, bo)


def kernel(x, edge_index, Wr0, Wi0, br0, bi0, Wr, Wi, br, bi, Wo, bo):
    n, din = x.shape
    e = edge_index.shape[1]
    nl = Wr.shape[0]
    dout = Wo.shape[1]
    src = edge_index[0].astype(jnp.int32)
    dst = edge_index[1].astype(jnp.int32)

    grp = _TILES * _K * _SLAB
    epad = ((e + grp - 1) // grp) * grp
    pad = epad - e
    if pad:
        ar = jnp.arange(pad, dtype=jnp.int32)
        src_p = jnp.concatenate([src, ar % n])
        dst_p = jnp.concatenate([dst, n + (ar % 8)])
    else:
        src_p, dst_p = src, dst
    gsrc = (src_p[None, :] +
            (jnp.arange(4, dtype=jnp.int32) * n)[:, None]).reshape(-1)

    w0c = jnp.concatenate([Wr0, Wi0], axis=1)
    biases = [jnp.concatenate([br0, bi0]).reshape(4, 1, 128)]
    bks = []
    for kk in range(nl):
        ws = 0.5 * (Wr[kk] + Wr[kk].T)
        wa = 0.5 * (Wi[kk] - Wi[kk].T)
        bks.append(jnp.concatenate([
            jnp.concatenate([ws, wa], axis=1),
            jnp.concatenate([-wa, ws], axis=1),
        ], axis=0))
        biases.append(jnp.concatenate([br[kk], bi[kk]]).reshape(4, 1, 128))

    deg16 = _sc_deg(dst, n=n, e=e).reshape(2, n, 128)
    hs, dinv = _tc_input(x, w0c, deg16, n=n, din=din)

    for kk in range(nl - 1):
        s_flat = _sc_prop(hs.reshape(4 * n, 128), gsrc, dst_p,
                          n=n, e=epad, nc=4)
        hs = _tc_hidden(s_flat.reshape(4, n, 128), hs, dinv,
                        biases[kk], bks[kk], n=n)

    s_flat = _sc_prop(hs.reshape(4 * n, 128), gsrc, dst_p,
                      n=n, e=epad, nc=4)
    hs = _tc_hidden(s_flat.reshape(4, n, 128), hs, dinv,
                    biases[nl - 1], bks[nl - 1], n=n)

    s2 = _sc_prop(hs.reshape(4 * n, 128), gsrc, dst_p,
                  n=n, e=epad, nc=2)
    out = _tc_output(s2.reshape(2, n, 128), hs[:2], dinv,
                     biases[nl].reshape(4, 1, 128)[:2], Wo,
                     bo.reshape(1, dout), n=n, dout=dout)
    return out


# speedup vs baseline: 1.0532x; 1.0532x over previous
import functools

import jax
import jax.numpy as jnp
from jax import lax
from jax.experimental import pallas as pl
from jax.experimental.pallas import tpu as pltpu
from jax.experimental.pallas import tpu_sc as plsc

_LANES = 16
_TILES = 16
_BN = 400


def _zero_rows(ref, nrows, ncols):
    zv = jnp.zeros((_LANES,), jnp.float32)

    def body(i, _):
        for t in range(ncols // _LANES):
            ref[i, pl.ds(t * _LANES, _LANES)] = zv
        return 0

    lax.fori_loop(0, nrows, body, 0)


def _node_partition(n):
    npt = (n // _TILES) & ~7
    tail = n - npt * _TILES
    return npt, tail


def _sliced_copy(src_fn, dst_fn, total, chunk):
    for b in range(total // chunk):
        pltpu.sync_copy(src_fn(b * chunk, chunk), dst_fn(b * chunk, chunk))
    rem = total % chunk
    if rem:
        off = (total // chunk) * chunk
        pltpu.sync_copy(src_fn(off, rem), dst_fn(off, rem))


def _fill_rows(ref, nrows, ncols, value):
    fv = jnp.full((_LANES,), value, jnp.float32)

    def body(i, _):
        for t in range(ncols // _LANES):
            ref[i, pl.ds(t * _LANES, _LANES)] = fv
        return 0

    lax.fori_loop(0, nrows, body, 0)


@functools.partial(jax.jit, static_argnames=("n", "e"))
def _sc_deg(dst, *, n, e):
    k = 80
    assert e % (2 * _TILES * k) == 0
    ept = e // (2 * _TILES)
    nblk = ept // k
    npt, tail = _node_partition(n)
    mesh = plsc.VectorSubcoreMesh(core_axis_name="c", subcore_axis_name="s",
                                  num_cores=2, num_subcores=_TILES)

    @functools.partial(
        pl.kernel,
        out_type=jax.ShapeDtypeStruct((2 * n, 128), jnp.float32),
        mesh=mesh,
        scratch_types=[
            pltpu.VMEM((k, 128), jnp.float32),
            pltpu.VMEM((k, 128), jnp.float32),
            pltpu.VMEM((8 * k,), jnp.int32),
            pltpu.VMEM_SHARED((n, 128), jnp.float32),
            pltpu.SemaphoreType.DMA((8,)),
            pltpu.SemaphoreType.DMA((4,)),
        ],
    )
    def deg_kernel(dst_hbm, out_hbm, ones, zrows, didx, acc, isem, ssem):
        c = lax.axis_index("c")
        s = lax.axis_index("s")
        _fill_rows(ones, k, 128, 1.0)
        _zero_rows(zrows, k, 128)
        tbase = c * (e // 2) + s * ept

        base = s * npt
        _sliced_copy(lambda o, sz: zrows.at[pl.ds(0, sz)],
                     lambda o, sz: acc.at[pl.ds(base + o, sz)], npt, k)

        @pl.when(s == _TILES - 1)
        def _():
            if tail:
                pltpu.sync_copy(zrows.at[pl.ds(0, tail)],
                                acc.at[pl.ds(npt * _TILES, tail)])

        plsc.subcore_barrier()

        def didx_at(slot):
            return didx.at[pl.ds(slot * k, k)]

        def fire_idx(b):
            pltpu.async_copy(dst_hbm.at[pl.ds(tbase + b * k, k)],
                             didx_at(lax.rem(b, 8)), isem.at[lax.rem(b, 8)])

        def wait_idx(b):
            pltpu.make_async_copy(dst_hbm.at[pl.ds(tbase, k)],
                                  didx_at(lax.rem(b, 8)),
                                  isem.at[lax.rem(b, 8)]).wait()

        def wait_scat(b):
            pltpu.make_async_copy(ones, acc.at[didx_at(lax.rem(b, 8))],
                                  ssem.at[lax.rem(b, 4)]).wait()

        fire_idx(jnp.int32(0))
        fire_idx(jnp.int32(1))
        fire_idx(jnp.int32(2))

        def eblk(b, _):
            wait_idx(b)

            @pl.when(b >= 4)
            def _():
                wait_scat(b - 4)

            pltpu.async_copy(ones, acc.at[didx_at(lax.rem(b, 8))],
                             ssem.at[lax.rem(b, 4)], add=True)

            @pl.when(b + 3 < nblk)
            def _():
                fire_idx(b + 3)

            return 0

        lax.fori_loop(0, nblk, eblk, 0)
        for t in range(4):
            wait_scat(jnp.int32(nblk - 4 + t))
        plsc.subcore_barrier()
        pltpu.sync_copy(acc.at[pl.ds(base, npt)],
                        out_hbm.at[pl.ds(c * n + base, npt)])

        @pl.when(s == _TILES - 1)
        def _():
            if tail:
                pltpu.sync_copy(acc.at[pl.ds(npt * _TILES, tail)],
                                out_hbm.at[pl.ds(c * n + npt * _TILES, tail)])

    return deg_kernel(dst)


_K = 88
_NSLOT = 4
_SLAB = 8
_IRING = 3


@functools.partial(jax.jit, static_argnames=("n", "e", "nc"))
def _sc_prop(hs_flat, gsrc, dst, *, n, e, nc):
    k = _K
    ept = e // _TILES
    nblk = ept // k
    assert e % (_TILES * k) == 0 and nblk % _SLAB == 0
    nslab = nblk // _SLAB
    assert nslab >= 2
    slab_i = _SLAB * k
    npt, tail = _node_partition(n)
    cpq = nc // 2
    mesh = plsc.VectorSubcoreMesh(core_axis_name="c", subcore_axis_name="s",
                                  num_cores=2, num_subcores=_TILES)

    @functools.partial(
        pl.kernel,
        out_type=jax.ShapeDtypeStruct((nc * n, 128), jnp.float32),
        mesh=mesh,
        scratch_types=[
            [pltpu.VMEM((k, 128), jnp.float32) for _ in range(_NSLOT)],
            pltpu.VMEM((_IRING * slab_i,), jnp.int32),
            pltpu.VMEM((_IRING * slab_i,), jnp.int32),
            pltpu.VMEM_SHARED((n + 8, 128), jnp.float32),
            pltpu.SemaphoreType.DMA((_NSLOT,)),
            pltpu.SemaphoreType.DMA((_NSLOT,)),
            pltpu.SemaphoreType.DMA((_IRING,)),
            pltpu.SemaphoreType.DMA((_IRING,)),
        ],
    )
    def prop_kernel(hs_hbm, gsrc_hbm, dst_hbm, out_hbm,
                    rows, gbuf, dbuf, acc, gsem, ssem, gisem, disem):
        c = lax.axis_index("c")
        s = lax.axis_index("s")
        base = s * npt
        tbase = s * ept

        def gidx_at(ring, r):
            return gbuf.at[pl.ds(ring * slab_i + r * k, k)]

        def didx_at(ring, r):
            return dbuf.at[pl.ds(ring * slab_i + r * k, k)]

        def fire_gather(slot, ring, r):
            pltpu.async_copy(hs_hbm.at[gidx_at(ring, r)], rows[slot],
                             gsem.at[slot])

        def wait_gather(slot, ring, r):
            pltpu.make_async_copy(hs_hbm.at[gidx_at(ring, r)], rows[slot],
                                  gsem.at[slot]).wait()

        def fire_scatter(slot, ring, r):
            pltpu.async_copy(rows[slot], acc.at[didx_at(ring, r)],
                             ssem.at[slot], add=True)

        def wait_scatter(slot, ring, r):
            pltpu.make_async_copy(rows[slot], acc.at[didx_at(ring, r)],
                                  ssem.at[slot]).wait()

        for j in range(cpq):
            q = c * cpq + j
            qrow = q * n
            qsrc = q * e

            _zero_rows(rows[0], k, 128)
            _sliced_copy(lambda o, sz: rows[0].at[pl.ds(0, sz)],
                         lambda o, sz: acc.at[pl.ds(base + o, sz)], npt, k)

            @pl.when(s == _TILES - 1)
            def _():
                if tail:
                    pltpu.sync_copy(rows[0].at[pl.ds(0, tail)],
                                    acc.at[pl.ds(npt * _TILES, tail)])

            plsc.subcore_barrier()

            pltpu.sync_copy(gsrc_hbm.at[pl.ds(qsrc + tbase, slab_i)],
                            gbuf.at[pl.ds(0, slab_i)])
            pltpu.sync_copy(dst_hbm.at[pl.ds(tbase, slab_i)],
                            dbuf.at[pl.ds(0, slab_i)])
            fire_gather(0, 0, 0)
            fire_gather(1, 0, 1)

            def slab(g, _):
                ring = lax.rem(g, _IRING)
                nring = lax.rem(g + 1, _IRING)

                @pl.when(g < nslab - 1)
                def _():
                    off = (g + 1) * slab_i
                    pltpu.async_copy(
                        gsrc_hbm.at[pl.ds(qsrc + tbase + off, slab_i)],
                        gbuf.at[pl.ds(nring * slab_i, slab_i)],
                        gisem.at[nring])
                    pltpu.async_copy(
                        dst_hbm.at[pl.ds(tbase + off, slab_i)],
                        dbuf.at[pl.ds(nring * slab_i, slab_i)],
                        disem.at[nring])

                for jj in range(_SLAB):
                    t = jj % _NSLOT
                    sp = (t + 2) % _NSLOT
                    wait_gather(t, ring, jj)
                    fire_scatter(t, ring, jj)
                    if jj < 2:
                        @pl.when(g > 0)
                        def _():
                            wait_scatter(sp, ring, jj)
                        fire_gather(sp, ring, jj + 2)
                    elif jj < _SLAB - 2:
                        wait_scatter(sp, ring, jj)
                        fire_gather(sp, ring, jj + 2)
                    else:
                        wait_scatter(sp, ring, jj)
                        if jj == _SLAB - 2:
                            @pl.when(g < nslab - 1)
                            def _():
                                pltpu.make_async_copy(
                                    gsrc_hbm.at[pl.ds(qsrc + tbase, slab_i)],
                                    gbuf.at[pl.ds(nring * slab_i, slab_i)],
                                    gisem.at[nring]).wait()
                                pltpu.make_async_copy(
                                    dst_hbm.at[pl.ds(tbase, slab_i)],
                                    dbuf.at[pl.ds(nring * slab_i, slab_i)],
                                    disem.at[nring]).wait()

                        @pl.when(g < nslab - 1)
                        def _():
                            fire_gather(sp, nring, jj - (_SLAB - 2))
                return 0

            lax.fori_loop(0, nslab, slab, 0)
            wait_scatter(2, 0, 0)
            wait_scatter(3, 0, 0)
            plsc.subcore_barrier()
            pltpu.sync_copy(acc.at[pl.ds(base, npt)],
                            out_hbm.at[pl.ds(qrow + base, npt)])

            @pl.when(s == _TILES - 1)
            def _():
                if tail:
                    pltpu.sync_copy(acc.at[pl.ds(npt * _TILES, tail)],
                                    out_hbm.at[pl.ds(qrow + npt * _TILES, tail)])

            plsc.subcore_barrier()

    return prop_kernel(hs_flat, gsrc, dst)


def _tc_input(x, w0c, deg16, *, n, din):
    def body(x_ref, w_ref, deg_ref, hs_ref, dinv_ref):
        d = deg_ref[0, :, 0:1] + deg_ref[1, :, 0:1]
        dv = lax.rsqrt(jnp.maximum(d + 1.0, 1.0))
        h = jnp.dot(x_ref[...], w_ref[...], preferred_element_type=jnp.float32)
        hs_ref[0] = h * dv
        dinv_ref[...] = dv

    grid = (n // _BN, 4)
    return pl.pallas_call(
        body,
        grid=grid,
        in_specs=[
            pl.BlockSpec((_BN, din), lambda i, q: (i, 0)),
            pl.BlockSpec((din, 128), lambda i, q: (0, q)),
            pl.BlockSpec((2, _BN, 128), lambda i, q: (0, i, 0)),
        ],
        out_specs=[
            pl.BlockSpec((1, _BN, 128), lambda i, q: (q, i, 0)),
            pl.BlockSpec((_BN, 1), lambda i, q: (i, 0)),
        ],
        out_shape=[
            jax.ShapeDtypeStruct((4, n, 128), jnp.float32),
            jax.ShapeDtypeStruct((n, 1), jnp.float32),
        ],
    )(x, w0c, deg16)


def _tc_hidden(s4, hs4, dinv, bias4, bk, *, n):
    def body(s_ref, hs_ref, dinv_ref, b_ref, w_ref, out_ref):
        dv = dinv_ref[...]
        acc = jnp.zeros((_BN, 128), jnp.float32)
        for qi in range(4):
            f = dv * (s_ref[qi] + hs_ref[qi]) + b_ref[qi]
            acc += jnp.dot(f, w_ref[pl.ds(qi * 128, 128), :],
                           preferred_element_type=jnp.float32)
        out_ref[0] = acc * dv

    grid = (n // _BN, 4)
    return pl.pallas_call(
        body,
        grid=grid,
        in_specs=[
            pl.BlockSpec((4, _BN, 128), lambda i, qo: (0, i, 0)),
            pl.BlockSpec((4, _BN, 128), lambda i, qo: (0, i, 0)),
            pl.BlockSpec((_BN, 1), lambda i, qo: (i, 0)),
            pl.BlockSpec((4, 1, 128), lambda i, qo: (0, 0, 0)),
            pl.BlockSpec((512, 128), lambda i, qo: (0, qo)),
        ],
        out_specs=pl.BlockSpec((1, _BN, 128), lambda i, qo: (qo, i, 0)),
        out_shape=jax.ShapeDtypeStruct((4, n, 128), jnp.float32),
    )(s4, hs4, dinv, bias4, bk)


def _tc_output(s2, hs2, dinv, bias2, wo, bo, *, n, dout):
    def body(s_ref, hs_ref, dinv_ref, b_ref, w_ref, bo_ref, out_ref):
        dv = dinv_ref[...]
        acc = jnp.zeros((_BN, dout), jnp.float32) + bo_ref[...]
        for qi in range(2):
            f = dv * (s_ref[qi] + hs_ref[qi]) + b_ref[qi]
            acc += jnp.dot(f, w_ref[pl.ds(qi * 128, 128), :],
                           preferred_element_type=jnp.float32)
        out_ref[...] = acc

    grid = (n // _BN,)
    return pl.pallas_call(
        body,
        grid=grid,
        in_specs=[
            pl.BlockSpec((2, _BN, 128), lambda i: (0, i, 0)),
            pl.BlockSpec((2, _BN, 128), lambda i: (0, i, 0)),
            pl.BlockSpec((_BN, 1), lambda i: (i, 0)),
            pl.BlockSpec((2, 1, 128), lambda i: (0, 0, 0)),
            pl.BlockSpec((256, dout), lambda i: (0, 0)),
            pl.BlockSpec((1, dout), lambda i: (0, 0)),
        ],
        out_specs=pl.BlockSpec((_BN, dout), lambda i: (i, 0)),
        out_shape=jax.ShapeDtypeStruct((n, dout), jnp.float32),
    )(s2, hs2, dinv, bias2, wo, bo)


def kernel(x, edge_index, Wr0, Wi0, br0, bi0, Wr, Wi, br, bi, Wo, bo):
    n, din = x.shape
    e = edge_index.shape[1]
    nl = Wr.shape[0]
    dout = Wo.shape[1]
    src = edge_index[0].astype(jnp.int32)
    dst = edge_index[1].astype(jnp.int32)

    grp = _TILES * _K * _SLAB
    epad = ((e + grp - 1) // grp) * grp
    pad = epad - e
    if pad:
        ar = jnp.arange(pad, dtype=jnp.int32)
        src_p = jnp.concatenate([src, ar % n])
        dst_p = jnp.concatenate([dst, n + (ar % 8)])
    else:
        src_p, dst_p = src, dst
    gsrc = (src_p[None, :] +
            (jnp.arange(4, dtype=jnp.int32) * n)[:, None]).reshape(-1)

    w0c = jnp.concatenate([Wr0, Wi0], axis=1)
    biases = [jnp.concatenate([br0, bi0]).reshape(4, 1, 128)]
    bks = []
    for kk in range(nl):
        ws = 0.5 * (Wr[kk] + Wr[kk].T)
        wa = 0.5 * (Wi[kk] - Wi[kk].T)
        bks.append(jnp.concatenate([
            jnp.concatenate([ws, wa], axis=1),
            jnp.concatenate([-wa, ws], axis=1),
        ], axis=0))
        biases.append(jnp.concatenate([br[kk], bi[kk]]).reshape(4, 1, 128))

    deg16 = _sc_deg(dst, n=n, e=e).reshape(2, n, 128)
    hs, dinv = _tc_input(x, w0c, deg16, n=n, din=din)

    for kk in range(nl - 1):
        s_flat = _sc_prop(hs.reshape(4 * n, 128), gsrc, dst_p,
                          n=n, e=epad, nc=4)
        hs = _tc_hidden(s_flat.reshape(4, n, 128), hs, dinv,
                        biases[kk], bks[kk], n=n)

    s_flat = _sc_prop(hs.reshape(4 * n, 128), gsrc, dst_p,
                      n=n, e=epad, nc=4)
    hs = _tc_hidden(s_flat.reshape(4, n, 128), hs, dinv,
                    biases[nl - 1], bks[nl - 1], n=n)

    s2 = _sc_prop(hs.reshape(4 * n, 128), gsrc, dst_p,
                  n=n, e=epad, nc=2)
    out = _tc_output(s2.reshape(2, n, 128), hs[:2], dinv,
                     biases[nl].reshape(4, 1, 128)[:2], Wo,
                     bo.reshape(1, dout), n=n, dout=dout)
    return out
